# baseline (speedup 1.0000x reference)
import functools

import jax
import jax.numpy as jnp
from jax.experimental import pallas as pl
from jax.experimental.pallas import tpu as pltpu

_INTERPRET = False

F32 = jnp.float32
BF16 = jnp.bfloat16


def _conv_body(x_ref, w_ref, b_ref, *rest, kh, kw, rb, W, Cin, Cout, relu,
               off_h, off_w, has_res):
    if has_res:
        res_ref, o_ref = rest
    else:
        (o_ref,) = rest
    r = pl.program_id(1)
    acc = jnp.zeros((rb * W, Cout), F32)
    for di in range(kh):
        for dj in range(kw):
            xs = x_ref[0, pl.ds(r * rb + di + off_h, rb),
                       pl.ds(dj + off_w, W), :]
            acc = acc + jnp.dot(xs.reshape(rb * W, Cin).astype(BF16),
                                w_ref[di * kw + dj].astype(BF16),
                                preferred_element_type=F32)
    acc = acc + b_ref[...]
    if relu:
        acc = jnp.maximum(acc, 0.0)
    y = acc.reshape(rb, W, Cout)
    if has_res:
        y = y + res_ref[0]
    o_ref[0] = y


def _conv(xp, w, b, *, kh, kw, H, W, relu=False, res=None, off_h=0, off_w=0,
          rb=8):
    B, Hp, Wp, Cin = xp.shape
    Cout = w.shape[2]
    if b is None:
        b = jnp.zeros((1, Cout), F32)
    else:
        b = b.reshape(1, Cout).astype(F32)
    in_specs = [
        pl.BlockSpec((1, Hp, Wp, Cin), lambda bb, r: (bb, 0, 0, 0)),
        pl.BlockSpec((kh * kw, Cin, Cout), lambda bb, r: (0, 0, 0)),
        pl.BlockSpec((1, Cout), lambda bb, r: (0, 0)),
    ]
    args = [xp, w, b]
    if res is not None:
        in_specs.append(pl.BlockSpec((1, rb, W, Cout),
                                     lambda bb, r: (bb, r, 0, 0)))
        args.append(res)
    body = functools.partial(_conv_body, kh=kh, kw=kw, rb=rb, W=W, Cin=Cin,
                             Cout=Cout, relu=relu, off_h=off_h, off_w=off_w,
                             has_res=res is not None)
    return pl.pallas_call(
        body,
        grid=(B, H // rb),
        in_specs=in_specs,
        out_specs=pl.BlockSpec((1, rb, W, Cout), lambda bb, r: (bb, r, 0, 0)),
        out_shape=jax.ShapeDtypeStruct((B, H, W, Cout), F32),
        interpret=_INTERPRET,
    )(*args)


def _res_body(x_ref, w0_ref, w1_ref, b1_ref, res_ref, o_ref, *, rb, W, C):
    r = pl.program_id(1)
    acc = jnp.zeros((rb * W, C), F32)
    for di in range(3):
        for dj in range(3):
            xs = x_ref[0, pl.ds(r * rb + di, rb), pl.ds(dj, W), :]
            acc = acc + jnp.dot(xs.reshape(rb * W, C).astype(BF16),
                                w0_ref[di * 3 + dj].astype(BF16),
                                preferred_element_type=F32)
    a = jnp.maximum(acc, 0.0)
    y = jnp.dot(a.astype(BF16), w1_ref[...].astype(BF16),
                preferred_element_type=F32) + b1_ref[...]
    o_ref[0] = y.reshape(rb, W, C) + res_ref[0]


def _res_block(x, w0, w1, b1, *, rb=8):
    B, H, W, C = x.shape
    xp = jnp.pad(x, ((0, 0), (1, 1), (1, 1), (0, 0)))
    body = functools.partial(_res_body, rb=rb, W=W, C=C)
    return pl.pallas_call(
        body,
        grid=(B, H // rb),
        in_specs=[
            pl.BlockSpec((1, H + 2, W + 2, C), lambda bb, r: (bb, 0, 0, 0)),
            pl.BlockSpec((9, C, C), lambda bb, r: (0, 0, 0)),
            pl.BlockSpec((C, C), lambda bb, r: (0, 0)),
            pl.BlockSpec((1, C), lambda bb, r: (0, 0)),
            pl.BlockSpec((1, rb, W, C), lambda bb, r: (bb, r, 0, 0)),
        ],
        out_specs=pl.BlockSpec((1, rb, W, C), lambda bb, r: (bb, r, 0, 0)),
        out_shape=jax.ShapeDtypeStruct((B, H, W, C), F32),
        interpret=_INTERPRET,
    )(xp, w0, w1, b1.reshape(1, C).astype(F32), x)


def _vq_body(h_ref, emb_ref, embT_ref, q_ref, misc_ref, en2_ref, cnt_ref,
             loss_ref, *, BR, C, K, KT, nsteps):
    i = pl.program_id(0)
    nkt = K // KT

    @pl.when(i == 0)
    def _init():
        en2_ref[...] = jnp.sum(embT_ref[...] * embT_ref[...], axis=0,
                               keepdims=True)
        cnt_ref[...] = jnp.zeros((1, K), F32)
        loss_ref[...] = jnp.zeros((1, 128), F32)

    h = h_ref[...]
    m = jnp.full((BR, 1), jnp.inf, F32)
    idx = jnp.zeros((BR, 1), jnp.int32)
    for kt in range(nkt):
        s = jnp.dot(h.astype(BF16),
                    embT_ref[:, kt * KT:(kt + 1) * KT].astype(BF16),
                    preferred_element_type=F32)
        s = en2_ref[:, kt * KT:(kt + 1) * KT] - 2.0 * s
        lane = jax.lax.broadcasted_iota(jnp.int32, (BR, KT), 1) + kt * KT
        mt = jnp.min(s, axis=1, keepdims=True)
        it = jnp.min(jnp.where(s == mt, lane, K), axis=1, keepdims=True)
        upd = mt < m
        m = jnp.where(upd, mt, m)
        idx = jnp.where(upd, it, idx)

    q = jnp.zeros((BR, C), F32)
    for kt in range(nkt):
        lane = jax.lax.broadcasted_iota(jnp.int32, (BR, KT), 1) + kt * KT
        oh = (lane == idx).astype(F32)
        q = q + jnp.dot(oh.astype(BF16),
                        emb_ref[kt * KT:(kt + 1) * KT, :].astype(BF16),
                        preferred_element_type=F32)
        sl = slice(kt * KT, (kt + 1) * KT)
        cnt_ref[:, sl] = cnt_ref[:, sl] + jnp.sum(oh, axis=0, keepdims=True)
    q_ref[...] = q

    step_loss = jnp.sum((q - h) * (q - h))
    lane128 = jax.lax.broadcasted_iota(jnp.int32, (1, 128), 1)
    loss_ref[...] = loss_ref[...] + jnp.where(lane128 == 0, step_loss, 0.0)

    @pl.when(i == nsteps - 1)
    def _fin():
        N = BR * nsteps
        probs = cnt_ref[...] / N
        ent = -jnp.sum(probs * jnp.log(probs + 1e-5))
        perp = jnp.exp(ent)
        loss = 0.25 * jnp.sum(loss_ref[...]) / (N * C)
        misc_ref[...] = jnp.where(lane128 == 0, loss,
                                  jnp.where(lane128 == 1, perp, 0.0))


def _vq(h_flat, emb):
    N, C = h_flat.shape
    K = emb.shape[0]
    BR = 448
    nsteps = N // BR
    body = functools.partial(_vq_body, BR=BR, C=C, K=K, KT=128, nsteps=nsteps)
    q, misc = pl.pallas_call(
        body,
        grid=(nsteps,),
        in_specs=[
            pl.BlockSpec((BR, C), lambda r: (r, 0)),
            pl.BlockSpec((K, C), lambda r: (0, 0)),
            pl.BlockSpec((C, K), lambda r: (0, 0)),
        ],
        out_specs=[
            pl.BlockSpec((BR, C), lambda r: (r, 0)),
            pl.BlockSpec((1, 128), lambda r: (0, 0)),
        ],
        out_shape=[
            jax.ShapeDtypeStruct((N, C), F32),
            jax.ShapeDtypeStruct((1, 128), F32),
        ],
        scratch_shapes=[
            pltpu.VMEM((1, K), F32),
            pltpu.VMEM((1, K), F32),
            pltpu.VMEM((1, 128), F32),
        ],
        interpret=_INTERPRET,
    )(h_flat, emb, emb.T)
    return q, misc[0, 0], misc[0, 1]


def _w_s1(w):
    O, I, kh, kw = w.shape
    return jnp.transpose(w, (2, 3, 1, 0)).reshape(kh * kw, I, O)


def _w_s2d(w):
    O, I, _, _ = w.shape
    w6 = w.reshape(O, I, 2, 2, 2, 2)
    wt = jnp.transpose(w6, (2, 4, 3, 5, 1, 0))
    return wt.reshape(4, 4 * I, O)


def _s2d(x):
    B, H2, W2, C = x.shape
    y = x.reshape(B, H2 // 2, 2, W2 // 2, 2, C)
    return jnp.transpose(y, (0, 1, 3, 2, 4, 5)).reshape(B, H2 // 2, W2 // 2,
                                                        4 * C)


def _w_convT_phase(w, qh, qw):
    hi = jnp.array([3 - qh, 1 - qh])
    wi = jnp.array([3 - qw, 1 - qw])
    sub = w[:, :, hi, :][:, :, :, wi]
    return jnp.transpose(sub, (2, 3, 0, 1)).reshape(4, w.shape[0], w.shape[1])


def _w_convT_packed(w):
    I, O, _, _ = w.shape
    w3 = jnp.zeros((3, 3, I, 4 * O), F32)
    for qh in (0, 1):
        for qw in (0, 1):
            for ah in (0, 1):
                for aw in (0, 1):
                    di, dj = qh + ah, qw + aw
                    th, tw = 3 - qh - 2 * ah, 3 - qw - 2 * aw
                    c0 = (qh * 2 + qw) * O
                    w3 = w3.at[di, dj, :, c0:c0 + O].set(w[:, :, th, tw])
    return w3.reshape(9, I, 4 * O)


def _d2s(parts, B, H, W, C):
    y = jnp.stack([parts[0][0], parts[0][1], parts[1][0], parts[1][1]],
                  axis=3)
    y = y.reshape(B, H, W, 2, 2, C)
    return jnp.transpose(y, (0, 1, 3, 2, 4, 5)).reshape(B, 2 * H, 2 * W, C)


def kernel(x, enc_w0, enc_b0, enc_w1, enc_b1, enc_w2, enc_b2,
           e0w0, e0w1, e0b1, e1w0, e1w1, e1b1, emb,
           dec_w, dec_b, d0w0, d0w1, d0b1, d1w0, d1w1, d1b1,
           tw0, tb0, tw1, tb1):
    B = x.shape[0]
    xh = jnp.transpose(x, (0, 2, 3, 1))
    xp = jnp.pad(xh, ((0, 0), (1, 1), (1, 1), (0, 0)))
    h = _conv(_s2d(xp), _w_s2d(enc_w0), enc_b0, kh=2, kw=2, H=112, W=112,
              relu=True)
    hp = jnp.pad(h, ((0, 0), (1, 1), (1, 1), (0, 0)))
    h = _conv(_s2d(hp), _w_s2d(enc_w1), enc_b1, kh=2, kw=2, H=56, W=56,
              relu=True)
    hp = jnp.pad(h, ((0, 0), (1, 1), (1, 1), (0, 0)))
    h = _conv(hp, _w_s1(enc_w2), enc_b2, kh=3, kw=3, H=56, W=56, relu=True)
    h = _res_block(h, _w_s1(e0w0), e0w1[:, :, 0, 0].T, e0b1)
    h = _res_block(h, _w_s1(e1w0), e1w1[:, :, 0, 0].T, e1b1)

    C = h.shape[3]
    q, loss, perp = _vq(h.reshape(-1, C), emb)
    q = q.reshape(B, 56, 56, C)

    qp = jnp.pad(q, ((0, 0), (1, 1), (1, 1), (0, 0)))
    g = _conv(qp, _w_s1(dec_w), dec_b, kh=3, kw=3, H=56, W=56)
    g = _res_block(g, _w_s1(d0w0), d0w1[:, :, 0, 0].T, d0b1)
    g = _res_block(g, _w_s1(d1w0), d1w1[:, :, 0, 0].T, d1b1)

    gp = jnp.pad(g, ((0, 0), (1, 1), (1, 1), (0, 0)))
    parts = [[None, None], [None, None]]
    for qh in (0, 1):
        for qw in (0, 1):
            parts[qh][qw] = _conv(gp, _w_convT_phase(tw0, qh, qw), tb0,
                                  kh=2, kw=2, H=56, W=56, relu=True,
                                  off_h=qh, off_w=qw)
    t = _d2s(parts, B, 56, 56, 128)

    tp = jnp.pad(t, ((0, 0), (1, 1), (1, 1), (0, 0)))
    bias12 = jnp.tile(tb1, 4)
    o = _conv(tp, _w_convT_packed(tw1), bias12, kh=3, kw=3, H=112, W=112)
    o = o.reshape(B, 112, 112, 2, 2, 3)
    o = jnp.transpose(o, (0, 1, 3, 2, 4, 5)).reshape(B, 224, 224, 3)
    g_out = jnp.transpose(o, (0, 3, 1, 2))

    return (g_out, loss, perp)


# speedup vs baseline: 1.1073x; 1.1073x over previous
import functools

import jax
import jax.numpy as jnp
from jax.experimental import pallas as pl
from jax.experimental.pallas import tpu as pltpu

_INTERPRET = False

F32 = jnp.float32
BF16 = jnp.bfloat16


def _conv_body(x_ref, w_ref, b_ref, *rest, kh, kw, rb, W, Cin, Cout, relu,
               off_h, off_w, has_res):
    if has_res:
        res_ref, o_ref = rest
    else:
        (o_ref,) = rest
    r = pl.program_id(1)
    acc = jnp.zeros((rb * W, Cout), F32)
    for di in range(kh):
        for dj in range(kw):
            xs = x_ref[0, pl.ds(r * rb + di + off_h, rb),
                       pl.ds(dj + off_w, W), :]
            acc = acc + jnp.dot(xs.reshape(rb * W, Cin).astype(BF16),
                                w_ref[di * kw + dj].astype(BF16),
                                preferred_element_type=F32)
    acc = acc + b_ref[...]
    if relu:
        acc = jnp.maximum(acc, 0.0)
    y = acc.reshape(rb, W, Cout)
    if has_res:
        y = y + res_ref[0]
    o_ref[0] = y


def _conv(xp, w, b, *, kh, kw, H, W, relu=False, res=None, off_h=0, off_w=0,
          rb=28):
    B, Hp, Wp, Cin = xp.shape
    Cout = w.shape[2]
    if b is None:
        b = jnp.zeros((1, Cout), F32)
    else:
        b = b.reshape(1, Cout).astype(F32)
    in_specs = [
        pl.BlockSpec((1, Hp, Wp, Cin), lambda bb, r: (bb, 0, 0, 0)),
        pl.BlockSpec((kh * kw, Cin, Cout), lambda bb, r: (0, 0, 0)),
        pl.BlockSpec((1, Cout), lambda bb, r: (0, 0)),
    ]
    args = [xp, w, b]
    if res is not None:
        in_specs.append(pl.BlockSpec((1, rb, W, Cout),
                                     lambda bb, r: (bb, r, 0, 0)))
        args.append(res)
    body = functools.partial(_conv_body, kh=kh, kw=kw, rb=rb, W=W, Cin=Cin,
                             Cout=Cout, relu=relu, off_h=off_h, off_w=off_w,
                             has_res=res is not None)
    return pl.pallas_call(
        body,
        grid=(B, H // rb),
        in_specs=in_specs,
        out_specs=pl.BlockSpec((1, rb, W, Cout), lambda bb, r: (bb, r, 0, 0)),
        out_shape=jax.ShapeDtypeStruct((B, H, W, Cout), F32),
        interpret=_INTERPRET,
    )(*args)


def _res_body(x_ref, w0_ref, w1_ref, b1_ref, res_ref, o_ref, *, rb, W, C):
    r = pl.program_id(1)
    acc = jnp.zeros((rb * W, C), F32)
    for di in range(3):
        for dj in range(3):
            xs = x_ref[0, pl.ds(r * rb + di, rb), pl.ds(dj, W), :]
            acc = acc + jnp.dot(xs.reshape(rb * W, C).astype(BF16),
                                w0_ref[di * 3 + dj].astype(BF16),
                                preferred_element_type=F32)
    a = jnp.maximum(acc, 0.0)
    y = jnp.dot(a.astype(BF16), w1_ref[...].astype(BF16),
                preferred_element_type=F32) + b1_ref[...]
    o_ref[0] = y.reshape(rb, W, C) + res_ref[0]


def _res_block(x, w0, w1, b1, *, rb=28):
    B, H, W, C = x.shape
    xp = jnp.pad(x, ((0, 0), (1, 1), (1, 1), (0, 0)))
    body = functools.partial(_res_body, rb=rb, W=W, C=C)
    return pl.pallas_call(
        body,
        grid=(B, H // rb),
        in_specs=[
            pl.BlockSpec((1, H + 2, W + 2, C), lambda bb, r: (bb, 0, 0, 0)),
            pl.BlockSpec((9, C, C), lambda bb, r: (0, 0, 0)),
            pl.BlockSpec((C, C), lambda bb, r: (0, 0)),
            pl.BlockSpec((1, C), lambda bb, r: (0, 0)),
            pl.BlockSpec((1, rb, W, C), lambda bb, r: (bb, r, 0, 0)),
        ],
        out_specs=pl.BlockSpec((1, rb, W, C), lambda bb, r: (bb, r, 0, 0)),
        out_shape=jax.ShapeDtypeStruct((B, H, W, C), F32),
        interpret=_INTERPRET,
    )(xp, w0, w1, b1.reshape(1, C).astype(F32), x)


def _vq_body(h_ref, emb_ref, embT_ref, q_ref, misc_ref, en2_ref, cnt_ref,
             loss_ref, *, BR, C, K, KT, nsteps):
    i = pl.program_id(0)
    nkt = K // KT

    @pl.when(i == 0)
    def _init():
        en2_ref[...] = jnp.sum(embT_ref[...] * embT_ref[...], axis=0,
                               keepdims=True)
        cnt_ref[...] = jnp.zeros((1, K), F32)
        loss_ref[...] = jnp.zeros((1, 128), F32)

    h = h_ref[...]
    m = jnp.full((BR, 1), jnp.inf, F32)
    idx = jnp.zeros((BR, 1), jnp.int32)
    for kt in range(nkt):
        s = jnp.dot(h.astype(BF16),
                    embT_ref[:, kt * KT:(kt + 1) * KT].astype(BF16),
                    preferred_element_type=F32)
        s = en2_ref[:, kt * KT:(kt + 1) * KT] - 2.0 * s
        lane = jax.lax.broadcasted_iota(jnp.int32, (BR, KT), 1) + kt * KT
        mt = jnp.min(s, axis=1, keepdims=True)
        it = jnp.min(jnp.where(s == mt, lane, K), axis=1, keepdims=True)
        upd = mt < m
        m = jnp.where(upd, mt, m)
        idx = jnp.where(upd, it, idx)

    q = jnp.zeros((BR, C), F32)
    for kt in range(nkt):
        lane = jax.lax.broadcasted_iota(jnp.int32, (BR, KT), 1) + kt * KT
        oh = (lane == idx).astype(F32)
        q = q + jnp.dot(oh.astype(BF16),
                        emb_ref[kt * KT:(kt + 1) * KT, :].astype(BF16),
                        preferred_element_type=F32)
        sl = slice(kt * KT, (kt + 1) * KT)
        cnt_ref[:, sl] = cnt_ref[:, sl] + jnp.sum(oh, axis=0, keepdims=True)
    q_ref[...] = q

    step_loss = jnp.sum((q - h) * (q - h))
    lane128 = jax.lax.broadcasted_iota(jnp.int32, (1, 128), 1)
    loss_ref[...] = loss_ref[...] + jnp.where(lane128 == 0, step_loss, 0.0)

    @pl.when(i == nsteps - 1)
    def _fin():
        N = BR * nsteps
        probs = cnt_ref[...] / N
        ent = -jnp.sum(probs * jnp.log(probs + 1e-5))
        perp = jnp.exp(ent)
        loss = 0.25 * jnp.sum(loss_ref[...]) / (N * C)
        misc_ref[...] = jnp.where(lane128 == 0, loss,
                                  jnp.where(lane128 == 1, perp, 0.0))


def _vq(h_flat, emb):
    N, C = h_flat.shape
    K = emb.shape[0]
    BR = 448
    nsteps = N // BR
    body = functools.partial(_vq_body, BR=BR, C=C, K=K, KT=128, nsteps=nsteps)
    q, misc = pl.pallas_call(
        body,
        grid=(nsteps,),
        in_specs=[
            pl.BlockSpec((BR, C), lambda r: (r, 0)),
            pl.BlockSpec((K, C), lambda r: (0, 0)),
            pl.BlockSpec((C, K), lambda r: (0, 0)),
        ],
        out_specs=[
            pl.BlockSpec((BR, C), lambda r: (r, 0)),
            pl.BlockSpec((1, 128), lambda r: (0, 0)),
        ],
        out_shape=[
            jax.ShapeDtypeStruct((N, C), F32),
            jax.ShapeDtypeStruct((1, 128), F32),
        ],
        scratch_shapes=[
            pltpu.VMEM((1, K), F32),
            pltpu.VMEM((1, K), F32),
            pltpu.VMEM((1, 128), F32),
        ],
        interpret=_INTERPRET,
    )(h_flat, emb, emb.T)
    return q, misc[0, 0], misc[0, 1]


def _t0_body(x_ref, w_ref, b_ref, o_ref, *, rb, W, Cin, Cout):
    r = pl.program_id(1)
    outs = []
    for p in range(4):
        qh, qw = p // 2, p % 2
        acc = jnp.zeros((rb * W, Cout), F32)
        for di in range(2):
            for dj in range(2):
                xs = x_ref[0, pl.ds(r * rb + di + qh, rb),
                           pl.ds(dj + qw, W), :]
                acc = acc + jnp.dot(xs.reshape(rb * W, Cin).astype(BF16),
                                    w_ref[p * 4 + di * 2 + dj].astype(BF16),
                                    preferred_element_type=F32)
        outs.append(jnp.maximum(acc + b_ref[...], 0.0))
    y = jnp.concatenate(outs, axis=1)
    o_ref[0] = y.reshape(rb, W, 4 * Cout)


def _t0_conv(xp, ws, b, *, H, W, rb=28):
    B, Hp, Wp, Cin = xp.shape
    Cout = ws.shape[2]
    body = functools.partial(_t0_body, rb=rb, W=W, Cin=Cin, Cout=Cout)
    return pl.pallas_call(
        body,
        grid=(B, H // rb),
        in_specs=[
            pl.BlockSpec((1, Hp, Wp, Cin), lambda bb, r: (bb, 0, 0, 0)),
            pl.BlockSpec((16, Cin, Cout), lambda bb, r: (0, 0, 0)),
            pl.BlockSpec((1, Cout), lambda bb, r: (0, 0)),
        ],
        out_specs=pl.BlockSpec((1, rb, W, 4 * Cout),
                               lambda bb, r: (bb, r, 0, 0)),
        out_shape=jax.ShapeDtypeStruct((B, H, W, 4 * Cout), F32),
        interpret=_INTERPRET,
    )(xp, ws, b.reshape(1, Cout).astype(F32))


def _w_s1(w):
    O, I, kh, kw = w.shape
    return jnp.transpose(w, (2, 3, 1, 0)).reshape(kh * kw, I, O)


def _w_s2d(w):
    O, I, _, _ = w.shape
    w6 = w.reshape(O, I, 2, 2, 2, 2)
    wt = jnp.transpose(w6, (2, 4, 3, 5, 1, 0))
    return wt.reshape(4, 4 * I, O)


def _s2d(x):
    B, H2, W2, C = x.shape
    y = x.reshape(B, H2 // 2, 2, W2 // 2, 2, C)
    return jnp.transpose(y, (0, 1, 3, 2, 4, 5)).reshape(B, H2 // 2, W2 // 2,
                                                        4 * C)


def _w_convT_phase(w, qh, qw):
    hi = jnp.array([3 - qh, 1 - qh])
    wi = jnp.array([3 - qw, 1 - qw])
    sub = w[:, :, hi, :][:, :, :, wi]
    return jnp.transpose(sub, (2, 3, 0, 1)).reshape(4, w.shape[0], w.shape[1])


def _w_convT_packed(w):
    I, O, _, _ = w.shape
    w3 = jnp.zeros((3, 3, I, 4 * O), F32)
    for qh in (0, 1):
        for qw in (0, 1):
            for ah in (0, 1):
                for aw in (0, 1):
                    di, dj = qh + ah, qw + aw
                    th, tw = 3 - qh - 2 * ah, 3 - qw - 2 * aw
                    c0 = (qh * 2 + qw) * O
                    w3 = w3.at[di, dj, :, c0:c0 + O].set(w[:, :, th, tw])
    return w3.reshape(9, I, 4 * O)


def _d2s(parts, B, H, W, C):
    y = jnp.stack([parts[0][0], parts[0][1], parts[1][0], parts[1][1]],
                  axis=3)
    y = y.reshape(B, H, W, 2, 2, C)
    return jnp.transpose(y, (0, 1, 3, 2, 4, 5)).reshape(B, 2 * H, 2 * W, C)


def kernel(x, enc_w0, enc_b0, enc_w1, enc_b1, enc_w2, enc_b2,
           e0w0, e0w1, e0b1, e1w0, e1w1, e1b1, emb,
           dec_w, dec_b, d0w0, d0w1, d0b1, d1w0, d1w1, d1b1,
           tw0, tb0, tw1, tb1):
    B = x.shape[0]
    xh = jnp.transpose(x, (0, 2, 3, 1))
    xp = jnp.pad(xh, ((0, 0), (1, 1), (1, 1), (0, 0)))
    h = _conv(_s2d(xp), _w_s2d(enc_w0), enc_b0, kh=2, kw=2, H=112, W=112,
              relu=True)
    hp = jnp.pad(h, ((0, 0), (1, 1), (1, 1), (0, 0)))
    h = _conv(_s2d(hp), _w_s2d(enc_w1), enc_b1, kh=2, kw=2, H=56, W=56,
              relu=True)
    hp = jnp.pad(h, ((0, 0), (1, 1), (1, 1), (0, 0)))
    h = _conv(hp, _w_s1(enc_w2), enc_b2, kh=3, kw=3, H=56, W=56, relu=True)
    h = _res_block(h, _w_s1(e0w0), e0w1[:, :, 0, 0].T, e0b1)
    h = _res_block(h, _w_s1(e1w0), e1w1[:, :, 0, 0].T, e1b1)

    C = h.shape[3]
    q, loss, perp = _vq(h.reshape(-1, C), emb)
    q = q.reshape(B, 56, 56, C)

    qp = jnp.pad(q, ((0, 0), (1, 1), (1, 1), (0, 0)))
    g = _conv(qp, _w_s1(dec_w), dec_b, kh=3, kw=3, H=56, W=56)
    g = _res_block(g, _w_s1(d0w0), d0w1[:, :, 0, 0].T, d0b1)
    g = _res_block(g, _w_s1(d1w0), d1w1[:, :, 0, 0].T, d1b1)

    gp = jnp.pad(g, ((0, 0), (1, 1), (1, 1), (0, 0)))
    ws = jnp.concatenate([_w_convT_phase(tw0, qh, qw)
                          for qh in (0, 1) for qw in (0, 1)], axis=0)
    y4 = _t0_conv(gp, ws, tb0, H=56, W=56)
    y4 = y4.reshape(B, 56, 56, 2, 2, 128)
    t = jnp.transpose(y4, (0, 1, 3, 2, 4, 5)).reshape(B, 112, 112, 128)

    tp = jnp.pad(t, ((0, 0), (1, 1), (1, 1), (0, 0)))
    bias12 = jnp.tile(tb1, 4)
    o = _conv(tp, _w_convT_packed(tw1), bias12, kh=3, kw=3, H=112, W=112)
    o = o.reshape(B, 112, 112, 2, 2, 3)
    o = jnp.transpose(o, (0, 1, 3, 2, 4, 5)).reshape(B, 224, 224, 3)
    g_out = jnp.transpose(o, (0, 3, 1, 2))

    return (g_out, loss, perp)


# speedup vs baseline: 1.1083x; 1.0009x over previous
import functools

import jax
import jax.numpy as jnp
from jax.experimental import pallas as pl
from jax.experimental.pallas import tpu as pltpu

_INTERPRET = False

F32 = jnp.float32
BF16 = jnp.bfloat16


def _conv_body(x_ref, w_ref, b_ref, *rest, kh, kw, rb, W, Cin, Cout, relu,
               off_h, off_w, has_res):
    if has_res:
        res_ref, o_ref = rest
    else:
        (o_ref,) = rest
    r = pl.program_id(1)
    acc = jnp.zeros((rb * W, Cout), F32)
    for di in range(kh):
        for dj in range(kw):
            xs = x_ref[0, pl.ds(r * rb + di + off_h, rb),
                       pl.ds(dj + off_w, W), :]
            acc = acc + jnp.dot(xs.reshape(rb * W, Cin).astype(BF16),
                                w_ref[di * kw + dj].astype(BF16),
                                preferred_element_type=F32)
    acc = acc + b_ref[...]
    if relu:
        acc = jnp.maximum(acc, 0.0)
    y = acc.reshape(rb, W, Cout)
    if has_res:
        y = y + res_ref[0]
    o_ref[0] = y


def _conv(xp, w, b, *, kh, kw, H, W, relu=False, res=None, off_h=0, off_w=0,
          rb=56):
    B, Hp, Wp, Cin = xp.shape
    Cout = w.shape[2]
    if b is None:
        b = jnp.zeros((1, Cout), F32)
    else:
        b = b.reshape(1, Cout).astype(F32)
    in_specs = [
        pl.BlockSpec((1, Hp, Wp, Cin), lambda bb, r: (bb, 0, 0, 0)),
        pl.BlockSpec((kh * kw, Cin, Cout), lambda bb, r: (0, 0, 0)),
        pl.BlockSpec((1, Cout), lambda bb, r: (0, 0)),
    ]
    args = [xp, w, b]
    if res is not None:
        in_specs.append(pl.BlockSpec((1, rb, W, Cout),
                                     lambda bb, r: (bb, r, 0, 0)))
        args.append(res)
    body = functools.partial(_conv_body, kh=kh, kw=kw, rb=rb, W=W, Cin=Cin,
                             Cout=Cout, relu=relu, off_h=off_h, off_w=off_w,
                             has_res=res is not None)
    return pl.pallas_call(
        body,
        grid=(B, H // rb),
        in_specs=in_specs,
        out_specs=pl.BlockSpec((1, rb, W, Cout), lambda bb, r: (bb, r, 0, 0)),
        out_shape=jax.ShapeDtypeStruct((B, H, W, Cout), F32),
        interpret=_INTERPRET,
    )(*args)


def _res_body(x_ref, w0_ref, w1_ref, b1_ref, res_ref, o_ref, *, rb, W, C):
    r = pl.program_id(1)
    acc = jnp.zeros((rb * W, C), F32)
    for di in range(3):
        for dj in range(3):
            xs = x_ref[0, pl.ds(r * rb + di, rb), pl.ds(dj, W), :]
            acc = acc + jnp.dot(xs.reshape(rb * W, C).astype(BF16),
                                w0_ref[di * 3 + dj].astype(BF16),
                                preferred_element_type=F32)
    a = jnp.maximum(acc, 0.0)
    y = jnp.dot(a.astype(BF16), w1_ref[...].astype(BF16),
                preferred_element_type=F32) + b1_ref[...]
    o_ref[0] = y.reshape(rb, W, C) + res_ref[0]


def _res_block(x, w0, w1, b1, *, rb=56):
    B, H, W, C = x.shape
    xp = jnp.pad(x, ((0, 0), (1, 1), (1, 1), (0, 0)))
    body = functools.partial(_res_body, rb=rb, W=W, C=C)
    return pl.pallas_call(
        body,
        grid=(B, H // rb),
        in_specs=[
            pl.BlockSpec((1, H + 2, W + 2, C), lambda bb, r: (bb, 0, 0, 0)),
            pl.BlockSpec((9, C, C), lambda bb, r: (0, 0, 0)),
            pl.BlockSpec((C, C), lambda bb, r: (0, 0)),
            pl.BlockSpec((1, C), lambda bb, r: (0, 0)),
            pl.BlockSpec((1, rb, W, C), lambda bb, r: (bb, r, 0, 0)),
        ],
        out_specs=pl.BlockSpec((1, rb, W, C), lambda bb, r: (bb, r, 0, 0)),
        out_shape=jax.ShapeDtypeStruct((B, H, W, C), F32),
        interpret=_INTERPRET,
    )(xp, w0, w1, b1.reshape(1, C).astype(F32), x)


def _vq_body(h_ref, emb_ref, embT_ref, q_ref, misc_ref, en2_ref, cnt_ref,
             loss_ref, *, BR, C, K, KT, nsteps):
    i = pl.program_id(0)
    nkt = K // KT

    @pl.when(i == 0)
    def _init():
        en2_ref[...] = jnp.sum(embT_ref[...] * embT_ref[...], axis=0,
                               keepdims=True)
        cnt_ref[...] = jnp.zeros((1, K), F32)
        loss_ref[...] = jnp.zeros((1, 128), F32)

    h = h_ref[...]
    m = jnp.full((BR, 1), jnp.inf, F32)
    idx = jnp.zeros((BR, 1), jnp.int32)
    for kt in range(nkt):
        s = jnp.dot(h.astype(BF16),
                    embT_ref[:, kt * KT:(kt + 1) * KT].astype(BF16),
                    preferred_element_type=F32)
        s = en2_ref[:, kt * KT:(kt + 1) * KT] - 2.0 * s
        lane = jax.lax.broadcasted_iota(jnp.int32, (BR, KT), 1) + kt * KT
        mt = jnp.min(s, axis=1, keepdims=True)
        it = jnp.min(jnp.where(s == mt, lane, K), axis=1, keepdims=True)
        upd = mt < m
        m = jnp.where(upd, mt, m)
        idx = jnp.where(upd, it, idx)

    q = jnp.zeros((BR, C), F32)
    for kt in range(nkt):
        lane = jax.lax.broadcasted_iota(jnp.int32, (BR, KT), 1) + kt * KT
        oh = (lane == idx).astype(F32)
        q = q + jnp.dot(oh.astype(BF16),
                        emb_ref[kt * KT:(kt + 1) * KT, :].astype(BF16),
                        preferred_element_type=F32)
        sl = slice(kt * KT, (kt + 1) * KT)
        cnt_ref[:, sl] = cnt_ref[:, sl] + jnp.sum(oh, axis=0, keepdims=True)
    q_ref[...] = q

    step_loss = jnp.sum((q - h) * (q - h))
    lane128 = jax.lax.broadcasted_iota(jnp.int32, (1, 128), 1)
    loss_ref[...] = loss_ref[...] + jnp.where(lane128 == 0, step_loss, 0.0)

    @pl.when(i == nsteps - 1)
    def _fin():
        N = BR * nsteps
        probs = cnt_ref[...] / N
        ent = -jnp.sum(probs * jnp.log(probs + 1e-5))
        perp = jnp.exp(ent)
        loss = 0.25 * jnp.sum(loss_ref[...]) / (N * C)
        misc_ref[...] = jnp.where(lane128 == 0, loss,
                                  jnp.where(lane128 == 1, perp, 0.0))


def _vq(h_flat, emb):
    N, C = h_flat.shape
    K = emb.shape[0]
    BR = 448
    nsteps = N // BR
    body = functools.partial(_vq_body, BR=BR, C=C, K=K, KT=128, nsteps=nsteps)
    q, misc = pl.pallas_call(
        body,
        grid=(nsteps,),
        in_specs=[
            pl.BlockSpec((BR, C), lambda r: (r, 0)),
            pl.BlockSpec((K, C), lambda r: (0, 0)),
            pl.BlockSpec((C, K), lambda r: (0, 0)),
        ],
        out_specs=[
            pl.BlockSpec((BR, C), lambda r: (r, 0)),
            pl.BlockSpec((1, 128), lambda r: (0, 0)),
        ],
        out_shape=[
            jax.ShapeDtypeStruct((N, C), F32),
            jax.ShapeDtypeStruct((1, 128), F32),
        ],
        scratch_shapes=[
            pltpu.VMEM((1, K), F32),
            pltpu.VMEM((1, K), F32),
            pltpu.VMEM((1, 128), F32),
        ],
        interpret=_INTERPRET,
    )(h_flat, emb, emb.T)
    return q, misc[0, 0], misc[0, 1]


def _t0_body(x_ref, w_ref, b_ref, o_ref, *, rb, W, Cin, Cout):
    r = pl.program_id(1)
    outs = []
    for p in range(4):
        qh, qw = p // 2, p % 2
        acc = jnp.zeros((rb * W, Cout), F32)
        for di in range(2):
            for dj in range(2):
                xs = x_ref[0, pl.ds(r * rb + di + qh, rb),
                           pl.ds(dj + qw, W), :]
                acc = acc + jnp.dot(xs.reshape(rb * W, Cin).astype(BF16),
                                    w_ref[p * 4 + di * 2 + dj].astype(BF16),
                                    preferred_element_type=F32)
        outs.append(jnp.maximum(acc + b_ref[...], 0.0))
    y = jnp.concatenate(outs, axis=1)
    o_ref[0] = y.reshape(rb, W, 4 * Cout)


def _t0_conv(xp, ws, b, *, H, W, rb=56):
    B, Hp, Wp, Cin = xp.shape
    Cout = ws.shape[2]
    body = functools.partial(_t0_body, rb=rb, W=W, Cin=Cin, Cout=Cout)
    return pl.pallas_call(
        body,
        grid=(B, H // rb),
        in_specs=[
            pl.BlockSpec((1, Hp, Wp, Cin), lambda bb, r: (bb, 0, 0, 0)),
            pl.BlockSpec((16, Cin, Cout), lambda bb, r: (0, 0, 0)),
            pl.BlockSpec((1, Cout), lambda bb, r: (0, 0)),
        ],
        out_specs=pl.BlockSpec((1, rb, W, 4 * Cout),
                               lambda bb, r: (bb, r, 0, 0)),
        out_shape=jax.ShapeDtypeStruct((B, H, W, 4 * Cout), F32),
        interpret=_INTERPRET,
    )(xp, ws, b.reshape(1, Cout).astype(F32))


def _w_s1(w):
    O, I, kh, kw = w.shape
    return jnp.transpose(w, (2, 3, 1, 0)).reshape(kh * kw, I, O)


def _w_s2d(w):
    O, I, _, _ = w.shape
    w6 = w.reshape(O, I, 2, 2, 2, 2)
    wt = jnp.transpose(w6, (2, 4, 3, 5, 1, 0))
    return wt.reshape(4, 4 * I, O)


def _s2d(x):
    B, H2, W2, C = x.shape
    y = x.reshape(B, H2 // 2, 2, W2 // 2, 2, C)
    return jnp.transpose(y, (0, 1, 3, 2, 4, 5)).reshape(B, H2 // 2, W2 // 2,
                                                        4 * C)


def _w_convT_phase(w, qh, qw):
    hi = jnp.array([3 - qh, 1 - qh])
    wi = jnp.array([3 - qw, 1 - qw])
    sub = w[:, :, hi, :][:, :, :, wi]
    return jnp.transpose(sub, (2, 3, 0, 1)).reshape(4, w.shape[0], w.shape[1])


def _w_convT_packed(w):
    I, O, _, _ = w.shape
    w3 = jnp.zeros((3, 3, I, 4 * O), F32)
    for qh in (0, 1):
        for qw in (0, 1):
            for ah in (0, 1):
                for aw in (0, 1):
                    di, dj = qh + ah, qw + aw
                    th, tw = 3 - qh - 2 * ah, 3 - qw - 2 * aw
                    c0 = (qh * 2 + qw) * O
                    w3 = w3.at[di, dj, :, c0:c0 + O].set(w[:, :, th, tw])
    return w3.reshape(9, I, 4 * O)


def _d2s(parts, B, H, W, C):
    y = jnp.stack([parts[0][0], parts[0][1], parts[1][0], parts[1][1]],
                  axis=3)
    y = y.reshape(B, H, W, 2, 2, C)
    return jnp.transpose(y, (0, 1, 3, 2, 4, 5)).reshape(B, 2 * H, 2 * W, C)


def kernel(x, enc_w0, enc_b0, enc_w1, enc_b1, enc_w2, enc_b2,
           e0w0, e0w1, e0b1, e1w0, e1w1, e1b1, emb,
           dec_w, dec_b, d0w0, d0w1, d0b1, d1w0, d1w1, d1b1,
           tw0, tb0, tw1, tb1):
    B = x.shape[0]
    xh = jnp.transpose(x, (0, 2, 3, 1))
    xp = jnp.pad(xh, ((0, 0), (1, 1), (1, 1), (0, 0)))
    h = _conv(_s2d(xp), _w_s2d(enc_w0), enc_b0, kh=2, kw=2, H=112, W=112,
              relu=True)
    hp = jnp.pad(h, ((0, 0), (1, 1), (1, 1), (0, 0)))
    h = _conv(_s2d(hp), _w_s2d(enc_w1), enc_b1, kh=2, kw=2, H=56, W=56,
              relu=True)
    hp = jnp.pad(h, ((0, 0), (1, 1), (1, 1), (0, 0)))
    h = _conv(hp, _w_s1(enc_w2), enc_b2, kh=3, kw=3, H=56, W=56, relu=True)
    h = _res_block(h, _w_s1(e0w0), e0w1[:, :, 0, 0].T, e0b1)
    h = _res_block(h, _w_s1(e1w0), e1w1[:, :, 0, 0].T, e1b1)

    C = h.shape[3]
    q, loss, perp = _vq(h.reshape(-1, C), emb)
    q = q.reshape(B, 56, 56, C)

    qp = jnp.pad(q, ((0, 0), (1, 1), (1, 1), (0, 0)))
    g = _conv(qp, _w_s1(dec_w), dec_b, kh=3, kw=3, H=56, W=56)
    g = _res_block(g, _w_s1(d0w0), d0w1[:, :, 0, 0].T, d0b1)
    g = _res_block(g, _w_s1(d1w0), d1w1[:, :, 0, 0].T, d1b1)

    gp = jnp.pad(g, ((0, 0), (1, 1), (1, 1), (0, 0)))
    ws = jnp.concatenate([_w_convT_phase(tw0, qh, qw)
                          for qh in (0, 1) for qw in (0, 1)], axis=0)
    y4 = _t0_conv(gp, ws, tb0, H=56, W=56)
    y4 = y4.reshape(B, 56, 56, 2, 2, 128)
    t = jnp.transpose(y4, (0, 1, 3, 2, 4, 5)).reshape(B, 112, 112, 128)

    tp = jnp.pad(t, ((0, 0), (1, 1), (1, 1), (0, 0)))
    bias12 = jnp.tile(tb1, 4)
    o = _conv(tp, _w_convT_packed(tw1), bias12, kh=3, kw=3, H=112, W=112)
    o = o.reshape(B, 112, 112, 2, 2, 3)
    o = jnp.transpose(o, (0, 1, 3, 2, 4, 5)).reshape(B, 224, 224, 3)
    g_out = jnp.transpose(o, (0, 3, 1, 2))

    return (g_out, loss, perp)
